# Initial kernel scaffold; baseline (speedup 1.0000x reference)
import jax
import jax.numpy as jnp
from jax.experimental import pallas as pl


def kernel(features, batch, edge_index, W, b, gamma, beta):
    raise NotImplementedError("write your pallas kernel here")


# speedup vs baseline: 3.5410x; 3.5410x over previous
import functools

import jax
import jax.numpy as jnp
from jax import lax
from jax.experimental import pallas as pl
from jax.experimental.pallas import tpu as pltpu
from jax.experimental.pallas import tpu_sc as plsc

N_NODES = 10000
D_IN = 128
D_OUT = 128
N_EDGES = 320000
BN_EPS = 1e-5

NC = 2
NS = 16
EC = 128
EPT = 10240
NCHUNK = EPT // EC
EPAD = EPT * NC * NS
ACC_ROWS = N_NODES + 16
ZROWS = 25
ROWS_PT = N_NODES // NS
NBCH = 78
BTAIL = N_NODES - NBCH * EC


def _sc_agg_gather(f, tgt, src, bat_main, bat_tail):
    mesh = plsc.VectorSubcoreMesh(core_axis_name="c", subcore_axis_name="s")

    @functools.partial(
        pl.kernel,
        out_type=[jax.ShapeDtypeStruct((N_NODES, D_IN), jnp.float32)] * 3,
        mesh=mesh,
        scratch_types=[
            pltpu.VMEM_SHARED((ACC_ROWS, D_IN), jnp.float32),
            pltpu.VMEM((NCHUNK, EC), jnp.int32),
            pltpu.VMEM((NCHUNK, EC), jnp.int32),
            pltpu.VMEM((EC, D_IN), jnp.float32),
            pltpu.VMEM((ZROWS, D_IN), jnp.float32),
            pltpu.VMEM((EC,), jnp.int32),
            pltpu.VMEM((BTAIL,), jnp.int32),
            pltpu.VMEM((BTAIL, D_IN), jnp.float32),
            pltpu.SemaphoreType.DMA,
        ],
    )
    def sc_kernel(f_h, tgt_h, src_h, batm_h, batt_h,
                  hf_h, haa_h, hab_h,
                  acc, tgt_v, src_v, rows, zbuf, bidx, bidx_t, gbuf_t,
                  sem):
        c = lax.axis_index("c")
        s = lax.axis_index("s")
        w = c * NS + s

        pltpu.sync_copy(tgt_h.at[c, s], tgt_v)
        pltpu.sync_copy(src_h.at[c, s], src_v)

        zv = jnp.zeros((16,), jnp.float32)
        for r in range(ZROWS):
            for k in range(D_IN // 16):
                zbuf[r, pl.ds(k * 16, 16)] = zv
        for k in range(ROWS_PT // ZROWS):
            pltpu.sync_copy(zbuf, acc.at[pl.ds(s * ROWS_PT + k * ZROWS, ZROWS)])
        plsc.subcore_barrier()

        def edge_body(j, carry):
            pltpu.async_copy(f_h.at[tgt_v.at[j]], rows, sem).wait()
            pltpu.sync_copy(rows, acc.at[src_v.at[j]], add=True)
            return carry

        lax.fori_loop(0, NCHUNK, edge_body, 0)
        plsc.subcore_barrier()

        for j2 in range(-(-NBCH // (NC * NS))):
            j = w + NC * NS * j2

            @pl.when(j < NBCH)
            def _():
                pltpu.sync_copy(batm_h.at[j], bidx)
                pltpu.async_copy(f_h.at[bidx], rows, sem).wait()
                pltpu.sync_copy(rows, hf_h.at[pl.ds(j * EC, EC)])

        for j2 in range(-(-NBCH // NS)):
            j = s + NS * j2

            @pl.when(j < NBCH)
            def _():
                pltpu.sync_copy(batm_h.at[j], bidx)
                pltpu.async_copy(acc.at[bidx], rows, sem).wait()

                @pl.when(c == 0)
                def _():
                    pltpu.sync_copy(rows, haa_h.at[pl.ds(j * EC, EC)])

                @pl.when(c == 1)
                def _():
                    pltpu.sync_copy(rows, hab_h.at[pl.ds(j * EC, EC)])

        @pl.when(s == NS - 1)
        def _():
            pltpu.sync_copy(batt_h, bidx_t)
            pltpu.async_copy(acc.at[bidx_t], gbuf_t, sem).wait()

            @pl.when(c == 0)
            def _():
                pltpu.sync_copy(gbuf_t, haa_h.at[pl.ds(NBCH * EC, BTAIL)])

            @pl.when(c == 1)
            def _():
                pltpu.sync_copy(gbuf_t, hab_h.at[pl.ds(NBCH * EC, BTAIL)])

        @pl.when((s == NS - 2) & (c == 0))
        def _():
            pltpu.sync_copy(batt_h, bidx_t)
            pltpu.async_copy(f_h.at[bidx_t], gbuf_t, sem).wait()
            pltpu.sync_copy(gbuf_t, hf_h.at[pl.ds(NBCH * EC, BTAIL)])

    return sc_kernel(f, tgt, src, bat_main, bat_tail)


_RB = 1000


def _tc_dense_body(hf, haa, hab, w, b, g, bt, o):
    dn = (((1,), (1,)), ((), ()))
    ha = haa[...] + hab[...]
    x = lax.dot_general(hf[...], w[...][:, :D_IN], dn,
                        preferred_element_type=jnp.float32)
    x += lax.dot_general(ha, w[...][:, D_IN:], dn,
                         preferred_element_type=jnp.float32)
    z = jnp.maximum(x + b[...], 0.0)
    scale = g[...] * lax.rsqrt(jnp.float32(1.0 + BN_EPS))
    z = z * scale + bt[...]
    nrm = jnp.sqrt(jnp.sum(z * z, axis=1, keepdims=True))
    o[...] = z / (nrm + 1e-6)


def _tc_dense(hf, haa, hab, w, b, g, bt):
    grid = (N_NODES // _RB,)
    row_spec = pl.BlockSpec((_RB, D_IN), lambda i: (i, 0))
    vec_spec = pl.BlockSpec((1, D_OUT), lambda i: (0, 0))
    return pl.pallas_call(
        _tc_dense_body,
        grid=grid,
        in_specs=[row_spec, row_spec, row_spec,
                  pl.BlockSpec((D_OUT, 2 * D_IN), lambda i: (0, 0)),
                  vec_spec, vec_spec, vec_spec],
        out_specs=pl.BlockSpec((_RB, D_OUT), lambda i: (i, 0)),
        out_shape=jax.ShapeDtypeStruct((N_NODES, D_OUT), jnp.float32),
    )(hf, haa, hab, w, b, g, bt)


def kernel(features, batch, edge_index, W, b, gamma, beta):
    f32 = jnp.float32
    i32 = jnp.int32
    pad = EPAD - N_EDGES
    src = jnp.concatenate(
        [edge_index[0].astype(i32), jnp.full((pad,), N_NODES, i32)])
    tgt = jnp.concatenate(
        [edge_index[1].astype(i32), jnp.zeros((pad,), i32)])
    src_r = src.reshape(NC, NS, NCHUNK, EC)
    tgt_r = tgt.reshape(NC, NS, NCHUNK, EC)
    bat = batch.astype(i32)
    bat_main = bat[:NBCH * EC].reshape(NBCH, EC)
    bat_tail = bat[NBCH * EC:]
    hf, haa, hab = _sc_agg_gather(features, tgt_r, src_r, bat_main, bat_tail)
    return _tc_dense(hf, haa, hab,
                     W.astype(f32), b.reshape(1, D_OUT).astype(f32),
                     gamma.reshape(1, D_OUT).astype(f32),
                     beta.reshape(1, D_OUT).astype(f32))


# speedup vs baseline: 9.6221x; 2.7173x over previous
import functools

import jax
import jax.numpy as jnp
from jax import lax
from jax.experimental import pallas as pl
from jax.experimental.pallas import tpu as pltpu
from jax.experimental.pallas import tpu_sc as plsc

N_NODES = 10000
D_IN = 128
D_OUT = 128
N_EDGES = 320000
BN_EPS = 1e-5

NC = 2
NS = 16
EC = 128
EPT = N_EDGES // (NC * NS)
NCH_E = EPT // EC
ETAIL = EPT - NCH_E * EC
ECPAD = NCH_E + 2
HB = 40
ZROWS = 25
ROWS_PT = N_NODES // NS
NBCH = 78
BTAIL = N_NODES - NBCH * EC


def _sc_agg_gather(f, tgt, src, bat_main, bat_tail):
    mesh = plsc.VectorSubcoreMesh(core_axis_name="c", subcore_axis_name="s")

    @functools.partial(
        pl.kernel,
        out_type=[jax.ShapeDtypeStruct((N_NODES, D_IN), jnp.float32)] * 3,
        mesh=mesh,
        scratch_types=[
            pltpu.VMEM_SHARED((N_NODES, D_IN), jnp.float32),
            pltpu.VMEM((HB, EC), jnp.int32),
            pltpu.VMEM((HB, EC), jnp.int32),
            pltpu.VMEM((ETAIL,), jnp.int32),
            pltpu.VMEM((ETAIL,), jnp.int32),
            pltpu.VMEM((2, EC, D_IN), jnp.float32),
            pltpu.VMEM((2, EC), jnp.int32),
            pltpu.VMEM((BTAIL,), jnp.int32),
            pltpu.SemaphoreType.DMA,
        ],
    )
    def sc_kernel(f_h, tgt_h, src_h, batm_h, batt_h,
                  hf_h, haa_h, hab_h,
                  acc, tgt_v, src_v, ttidx, stidx, rows, bidx, bidx_t, sem):
        c = lax.axis_index("c")
        s = lax.axis_index("s")
        w = c * NS + s

        zv = jnp.zeros((16,), jnp.float32)
        for r in range(ZROWS):
            for k in range(D_IN // 16):
                rows[0, r, pl.ds(k * 16, 16)] = zv
        for k in range(ROWS_PT // ZROWS):
            pltpu.sync_copy(rows.at[0, pl.ds(0, ZROWS)],
                            acc.at[pl.ds(s * ROWS_PT + k * ZROWS, ZROWS)])
        plsc.subcore_barrier()

        for half in range(2):
            nch = HB if half == 0 else NCH_E - HB
            pltpu.sync_copy(tgt_h.at[c, s, pl.ds(half * HB, HB)], tgt_v)
            pltpu.sync_copy(src_h.at[c, s, pl.ds(half * HB, HB)], src_v)
            pltpu.async_copy(f_h.at[tgt_v.at[0]], rows.at[0], sem)

            def edge_body(j, carry):
                p = lax.rem(j, 2)
                pltpu.make_async_copy(
                    f_h.at[tgt_v.at[j]], rows.at[p], sem).wait()

                @pl.when(j + 1 < nch)
                def _():
                    pltpu.async_copy(
                        f_h.at[tgt_v.at[j + 1]], rows.at[1 - p], sem)

                pltpu.sync_copy(rows.at[p], acc.at[src_v.at[j]], add=True)
                return carry

            lax.fori_loop(0, nch, edge_body, 0)

        pltpu.sync_copy(tgt_h.at[c, s, NCH_E, pl.ds(0, ETAIL)], ttidx)
        pltpu.sync_copy(src_h.at[c, s, NCH_E, pl.ds(0, ETAIL)], stidx)
        pltpu.async_copy(f_h.at[ttidx], rows.at[0, pl.ds(0, ETAIL)], sem).wait()
        pltpu.sync_copy(rows.at[0, pl.ds(0, ETAIL)], acc.at[stidx], add=True)
        plsc.subcore_barrier()

        nb_f = (NBCH - w + NC * NS - 1) // (NC * NS)

        @pl.when(nb_f > 0)
        def _():
            pltpu.sync_copy(batm_h.at[w], bidx.at[0])
            pltpu.async_copy(f_h.at[bidx.at[0]], rows.at[0], sem)

            def bf_body(i, carry):
                p = lax.rem(i, 2)
                j = w + NC * NS * i
                pltpu.make_async_copy(
                    f_h.at[bidx.at[p]], rows.at[p], sem).wait()

                @pl.when(i + 1 < nb_f)
                def _():
                    pltpu.sync_copy(batm_h.at[j + NC * NS], bidx.at[1 - p])
                    pltpu.async_copy(
                        f_h.at[bidx.at[1 - p]], rows.at[1 - p], sem)

                pltpu.sync_copy(rows.at[p], hf_h.at[pl.ds(j * EC, EC)])
                return carry

            lax.fori_loop(0, nb_f, bf_body, 0)

        @pl.when((c == 0) & (s == NS - 2))
        def _():
            pltpu.sync_copy(batt_h, bidx_t)
            pltpu.async_copy(
                f_h.at[bidx_t], rows.at[0, pl.ds(0, BTAIL)], sem).wait()
            pltpu.sync_copy(rows.at[0, pl.ds(0, BTAIL)],
                            hf_h.at[pl.ds(NBCH * EC, BTAIL)])

        nb_a = (NBCH - s + NS - 1) // NS

        def ba_body(i, carry):
            p = lax.rem(i, 2)
            j = s + NS * i
            pltpu.make_async_copy(acc.at[bidx.at[p]], rows.at[p], sem).wait()

            @pl.when(i + 1 < nb_a)
            def _():
                pltpu.sync_copy(batm_h.at[j + NS], bidx.at[1 - p])
                pltpu.async_copy(acc.at[bidx.at[1 - p]], rows.at[1 - p], sem)

            @pl.when(c == 0)
            def _():
                pltpu.sync_copy(rows.at[p], haa_h.at[pl.ds(j * EC, EC)])

            @pl.when(c == 1)
            def _():
                pltpu.sync_copy(rows.at[p], hab_h.at[pl.ds(j * EC, EC)])

            return carry

        pltpu.sync_copy(batm_h.at[s], bidx.at[0])
        pltpu.async_copy(acc.at[bidx.at[0]], rows.at[0], sem)
        lax.fori_loop(0, nb_a, ba_body, 0)

        @pl.when(s == NS - 1)
        def _():
            pltpu.sync_copy(batt_h, bidx_t)
            pltpu.async_copy(
                acc.at[bidx_t], rows.at[0, pl.ds(0, BTAIL)], sem).wait()

            @pl.when(c == 0)
            def _():
                pltpu.sync_copy(rows.at[0, pl.ds(0, BTAIL)],
                                haa_h.at[pl.ds(NBCH * EC, BTAIL)])

            @pl.when(c == 1)
            def _():
                pltpu.sync_copy(rows.at[0, pl.ds(0, BTAIL)],
                                hab_h.at[pl.ds(NBCH * EC, BTAIL)])

    return sc_kernel(f, tgt, src, bat_main, bat_tail)


_RB = 1000


def _tc_dense_body(hf, haa, hab, w, b, g, bt, o):
    dn = (((1,), (1,)), ((), ()))
    ha = haa[...] + hab[...]
    x = lax.dot_general(hf[...], w[...][:, :D_IN], dn,
                        preferred_element_type=jnp.float32)
    x += lax.dot_general(ha, w[...][:, D_IN:], dn,
                         preferred_element_type=jnp.float32)
    z = jnp.maximum(x + b[...], 0.0)
    scale = g[...] * lax.rsqrt(jnp.float32(1.0 + BN_EPS))
    z = z * scale + bt[...]
    nrm = jnp.sqrt(jnp.sum(z * z, axis=1, keepdims=True))
    o[...] = z / (nrm + 1e-6)


def _tc_dense(hf, haa, hab, w, b, g, bt):
    grid = (N_NODES // _RB,)
    row_spec = pl.BlockSpec((_RB, D_IN), lambda i: (i, 0))
    vec_spec = pl.BlockSpec((1, D_OUT), lambda i: (0, 0))
    return pl.pallas_call(
        _tc_dense_body,
        grid=grid,
        in_specs=[row_spec, row_spec, row_spec,
                  pl.BlockSpec((D_OUT, 2 * D_IN), lambda i: (0, 0)),
                  vec_spec, vec_spec, vec_spec],
        out_specs=pl.BlockSpec((_RB, D_OUT), lambda i: (i, 0)),
        out_shape=jax.ShapeDtypeStruct((N_NODES, D_OUT), jnp.float32),
    )(hf, haa, hab, w, b, g, bt)


def kernel(features, batch, edge_index, W, b, gamma, beta):
    f32 = jnp.float32
    i32 = jnp.int32
    per_tile_src = edge_index[0].astype(i32).reshape(NC * NS, EPT)
    per_tile_tgt = edge_index[1].astype(i32).reshape(NC * NS, EPT)
    padcols = ECPAD * EC - EPT
    per_tile_src = jnp.pad(per_tile_src, ((0, 0), (0, padcols)))
    per_tile_tgt = jnp.pad(per_tile_tgt, ((0, 0), (0, padcols)))
    src_r = per_tile_src.reshape(NC, NS, ECPAD, EC)
    tgt_r = per_tile_tgt.reshape(NC, NS, ECPAD, EC)
    bat = batch.astype(i32)
    bat_main = bat[:NBCH * EC].reshape(NBCH, EC)
    bat_tail = bat[NBCH * EC:]
    hf, haa, hab = _sc_agg_gather(features, tgt_r, src_r, bat_main, bat_tail)
    return _tc_dense(hf, haa, hab,
                     W.astype(f32), b.reshape(1, D_OUT).astype(f32),
                     gamma.reshape(1, D_OUT).astype(f32),
                     beta.reshape(1, D_OUT).astype(f32))


# speedup vs baseline: 10.9382x; 1.1368x over previous
import functools

import jax
import jax.numpy as jnp
from jax import lax
from jax.experimental import pallas as pl
from jax.experimental.pallas import tpu as pltpu
from jax.experimental.pallas import tpu_sc as plsc

N_NODES = 10000
D_IN = 128
D_OUT = 128
N_EDGES = 320000
BN_EPS = 1e-5

NC = 2
NS = 16
EC = 128
EPT = N_EDGES // (NC * NS)
NCH_E = EPT // EC
ETAIL = EPT - NCH_E * EC
ECPAD = NCH_E + 2
HB = 40
ZROWS = 25
ROWS_PT = N_NODES // NS
NBCH = 78
BTAIL = N_NODES - NBCH * EC


def _sc_agg_gather(f, tgt, src, bat_main, bat_tail):
    mesh = plsc.VectorSubcoreMesh(core_axis_name="c", subcore_axis_name="s")

    @functools.partial(
        pl.kernel,
        out_type=[jax.ShapeDtypeStruct((N_NODES, D_IN), jnp.float32)] * 3,
        mesh=mesh,
        scratch_types=[
            pltpu.VMEM_SHARED((N_NODES, D_IN), jnp.float32),
            pltpu.VMEM((HB, EC), jnp.int32),
            pltpu.VMEM((HB, EC), jnp.int32),
            pltpu.VMEM((ETAIL,), jnp.int32),
            pltpu.VMEM((ETAIL,), jnp.int32),
            pltpu.VMEM((2, EC, D_IN), jnp.float32),
            pltpu.VMEM((2, EC), jnp.int32),
            pltpu.VMEM((BTAIL,), jnp.int32),
            pltpu.SemaphoreType.DMA,
        ],
    )
    def sc_kernel(f_h, tgt_h, src_h, batm_h, batt_h,
                  hf_h, haa_h, hab_h,
                  acc, tgt_v, src_v, ttidx, stidx, rows, bidx, bidx_t, sem):
        c = lax.axis_index("c")
        s = lax.axis_index("s")
        w = c * NS + s

        zv = jnp.zeros((16,), jnp.float32)
        for r in range(ZROWS):
            for k in range(D_IN // 16):
                rows[0, r, pl.ds(k * 16, 16)] = zv
        for k in range(ROWS_PT // ZROWS):
            pltpu.sync_copy(rows.at[0, pl.ds(0, ZROWS)],
                            acc.at[pl.ds(s * ROWS_PT + k * ZROWS, ZROWS)])
        plsc.subcore_barrier()

        EH = EC // 2

        def _gather_halves(j, p):
            pltpu.async_copy(
                f_h.at[tgt_v.at[j, pl.ds(0, EH)]],
                rows.at[p, pl.ds(0, EH)], sem)
            pltpu.async_copy(
                f_h.at[tgt_v.at[j, pl.ds(EH, EH)]],
                rows.at[p, pl.ds(EH, EH)], sem)

        def _wait_halves(j, p):
            pltpu.make_async_copy(
                f_h.at[tgt_v.at[j, pl.ds(0, EH)]],
                rows.at[p, pl.ds(0, EH)], sem).wait()
            pltpu.make_async_copy(
                f_h.at[tgt_v.at[j, pl.ds(EH, EH)]],
                rows.at[p, pl.ds(EH, EH)], sem).wait()

        for half in range(2):
            nch = HB if half == 0 else NCH_E - HB
            pltpu.sync_copy(tgt_h.at[c, s, pl.ds(half * HB, HB)], tgt_v)
            pltpu.sync_copy(src_h.at[c, s, pl.ds(half * HB, HB)], src_v)
            _gather_halves(0, 0)
            if nch > 1:
                _gather_halves(1, 1)

            def edge_body(j, carry):
                p = lax.rem(j, 2)
                _wait_halves(j, p)
                pltpu.sync_copy(rows.at[p], acc.at[src_v.at[j]], add=True)

                @pl.when(j + 2 < nch)
                def _():
                    _gather_halves(j + 2, p)

                return carry

            lax.fori_loop(0, nch, edge_body, 0)

        pltpu.sync_copy(tgt_h.at[c, s, NCH_E, pl.ds(0, ETAIL)], ttidx)
        pltpu.sync_copy(src_h.at[c, s, NCH_E, pl.ds(0, ETAIL)], stidx)
        pltpu.async_copy(f_h.at[ttidx], rows.at[0, pl.ds(0, ETAIL)], sem).wait()
        pltpu.sync_copy(rows.at[0, pl.ds(0, ETAIL)], acc.at[stidx], add=True)
        plsc.subcore_barrier()

        nb_f = (NBCH - w + NC * NS - 1) // (NC * NS)

        @pl.when(nb_f > 0)
        def _():
            pltpu.sync_copy(batm_h.at[w], bidx.at[0])
            pltpu.async_copy(f_h.at[bidx.at[0]], rows.at[0], sem)

            def bf_body(i, carry):
                p = lax.rem(i, 2)
                j = w + NC * NS * i
                pltpu.make_async_copy(
                    f_h.at[bidx.at[p]], rows.at[p], sem).wait()

                @pl.when(i + 1 < nb_f)
                def _():
                    pltpu.sync_copy(batm_h.at[j + NC * NS], bidx.at[1 - p])
                    pltpu.async_copy(
                        f_h.at[bidx.at[1 - p]], rows.at[1 - p], sem)

                pltpu.sync_copy(rows.at[p], hf_h.at[pl.ds(j * EC, EC)])
                return carry

            lax.fori_loop(0, nb_f, bf_body, 0)

        @pl.when((c == 0) & (s == NS - 2))
        def _():
            pltpu.sync_copy(batt_h, bidx_t)
            pltpu.async_copy(
                f_h.at[bidx_t], rows.at[0, pl.ds(0, BTAIL)], sem).wait()
            pltpu.sync_copy(rows.at[0, pl.ds(0, BTAIL)],
                            hf_h.at[pl.ds(NBCH * EC, BTAIL)])

        nb_a = (NBCH - s + NS - 1) // NS

        def ba_body(i, carry):
            p = lax.rem(i, 2)
            j = s + NS * i
            pltpu.make_async_copy(acc.at[bidx.at[p]], rows.at[p], sem).wait()

            @pl.when(i + 1 < nb_a)
            def _():
                pltpu.sync_copy(batm_h.at[j + NS], bidx.at[1 - p])
                pltpu.async_copy(acc.at[bidx.at[1 - p]], rows.at[1 - p], sem)

            @pl.when(c == 0)
            def _():
                pltpu.sync_copy(rows.at[p], haa_h.at[pl.ds(j * EC, EC)])

            @pl.when(c == 1)
            def _():
                pltpu.sync_copy(rows.at[p], hab_h.at[pl.ds(j * EC, EC)])

            return carry

        pltpu.sync_copy(batm_h.at[s], bidx.at[0])
        pltpu.async_copy(acc.at[bidx.at[0]], rows.at[0], sem)
        lax.fori_loop(0, nb_a, ba_body, 0)

        @pl.when(s == NS - 1)
        def _():
            pltpu.sync_copy(batt_h, bidx_t)
            pltpu.async_copy(
                acc.at[bidx_t], rows.at[0, pl.ds(0, BTAIL)], sem).wait()

            @pl.when(c == 0)
            def _():
                pltpu.sync_copy(rows.at[0, pl.ds(0, BTAIL)],
                                haa_h.at[pl.ds(NBCH * EC, BTAIL)])

            @pl.when(c == 1)
            def _():
                pltpu.sync_copy(rows.at[0, pl.ds(0, BTAIL)],
                                hab_h.at[pl.ds(NBCH * EC, BTAIL)])

    return sc_kernel(f, tgt, src, bat_main, bat_tail)


_RB = 1000


def _tc_dense_body(hf, haa, hab, w, b, g, bt, o):
    dn = (((1,), (1,)), ((), ()))
    ha = haa[...] + hab[...]
    x = lax.dot_general(hf[...], w[...][:, :D_IN], dn,
                        preferred_element_type=jnp.float32)
    x += lax.dot_general(ha, w[...][:, D_IN:], dn,
                         preferred_element_type=jnp.float32)
    z = jnp.maximum(x + b[...], 0.0)
    scale = g[...] * lax.rsqrt(jnp.float32(1.0 + BN_EPS))
    z = z * scale + bt[...]
    nrm = jnp.sqrt(jnp.sum(z * z, axis=1, keepdims=True))
    o[...] = z / (nrm + 1e-6)


def _tc_dense(hf, haa, hab, w, b, g, bt):
    grid = (N_NODES // _RB,)
    row_spec = pl.BlockSpec((_RB, D_IN), lambda i: (i, 0))
    vec_spec = pl.BlockSpec((1, D_OUT), lambda i: (0, 0))
    return pl.pallas_call(
        _tc_dense_body,
        grid=grid,
        in_specs=[row_spec, row_spec, row_spec,
                  pl.BlockSpec((D_OUT, 2 * D_IN), lambda i: (0, 0)),
                  vec_spec, vec_spec, vec_spec],
        out_specs=pl.BlockSpec((_RB, D_OUT), lambda i: (i, 0)),
        out_shape=jax.ShapeDtypeStruct((N_NODES, D_OUT), jnp.float32),
    )(hf, haa, hab, w, b, g, bt)


def kernel(features, batch, edge_index, W, b, gamma, beta):
    f32 = jnp.float32
    i32 = jnp.int32
    per_tile_src = edge_index[0].astype(i32).reshape(NC * NS, EPT)
    per_tile_tgt = edge_index[1].astype(i32).reshape(NC * NS, EPT)
    padcols = ECPAD * EC - EPT
    per_tile_src = jnp.pad(per_tile_src, ((0, 0), (0, padcols)))
    per_tile_tgt = jnp.pad(per_tile_tgt, ((0, 0), (0, padcols)))
    src_r = per_tile_src.reshape(NC, NS, ECPAD, EC)
    tgt_r = per_tile_tgt.reshape(NC, NS, ECPAD, EC)
    bat = batch.astype(i32)
    bat_main = bat[:NBCH * EC].reshape(NBCH, EC)
    bat_tail = bat[NBCH * EC:]
    hf, haa, hab = _sc_agg_gather(features, tgt_r, src_r, bat_main, bat_tail)
    return _tc_dense(hf, haa, hab,
                     W.astype(f32), b.reshape(1, D_OUT).astype(f32),
                     gamma.reshape(1, D_OUT).astype(f32),
                     beta.reshape(1, D_OUT).astype(f32))
